# baseline (speedup 1.0000x reference)
import dataclasses
import functools

import jax
import jax.numpy as jnp
from jax import lax
from jax.experimental import pallas as pl
from jax.experimental.pallas import tpu as pltpu
from jax.experimental.pallas import tpu_sc as plsc

VOCAB = 100000
D = 128
B = 4096
L = 50

NC = 2
NS = 16
NW = NC * NS
LANES = 16
BLANES = 32
DP = D // 2
NCH = DP // LANES

SPT = B // NW
IPT = SPT * L
CH_S = 4
CH_I = CH_S * L
NCHUNK = SPT // CH_S
NBUF = 8

_SCALE = 1.0 / L


def _reduce_chunk(rows_v, out_v, chunk):
    zero = jnp.zeros((BLANES,), jnp.bfloat16)
    for s in range(CH_S):
        row0 = s * L
        init = (tuple(zero for _ in range(NCH)), tuple(zero for _ in range(NCH)))

        @plsc.parallel_loop(0, L // 2, carry=init)
        def accs(i, carry, _row0=row0):
            a, b = carry
            ra = _row0 + 2 * i
            a = tuple(
                a[c] + plsc.bitcast(
                    rows_v[ra, pl.ds(c * LANES, LANES)], jnp.bfloat16)
                for c in range(NCH)
            )
            b = tuple(
                b[c] + plsc.bitcast(
                    rows_v[ra + 1, pl.ds(c * LANES, LANES)], jnp.bfloat16)
                for c in range(NCH)
            )
            return (a, b)

        a, b = accs
        orow = chunk * CH_S + s
        scale = jnp.bfloat16(_SCALE)
        for c in range(NCH):
            out_v[orow, pl.ds(c * LANES, LANES)] = plsc.bitcast(
                (a[c] + b[c]) * scale, jnp.int32)


def kernel(token_ids, emb_table):
    flat_ids = token_ids.reshape(-1).astype(jnp.int32)
    table_pk = lax.bitcast_convert_type(
        emb_table.astype(jnp.bfloat16).reshape(VOCAB, DP, 2), jnp.int32)
    mesh = plsc.VectorSubcoreMesh(core_axis_name="c", subcore_axis_name="s")
    cp = pltpu.CompilerParams()
    if "needs_layout_passes" in pltpu.CompilerParams.__dataclass_fields__:
        cp = dataclasses.replace(cp, needs_layout_passes=False)
    if "use_tc_tiling_on_sc" in pltpu.CompilerParams.__dataclass_fields__:
        cp = dataclasses.replace(cp, use_tc_tiling_on_sc=False)

    @functools.partial(
        pl.kernel,
        out_type=jax.ShapeDtypeStruct((B, DP), jnp.int32),
        mesh=mesh,
        compiler_params=cp,
        scratch_types=[
            pltpu.VMEM((IPT,), jnp.int32),
            pltpu.VMEM((NBUF, CH_I, DP), jnp.int32),
            pltpu.VMEM((SPT, DP), jnp.int32),
        ]
        + [pltpu.SemaphoreType.DMA] * NBUF,
    )
    def tile_kernel(idx_hbm, table_hbm, out_hbm, idx_v, rows_v, out_v, *sems):
        wid = lax.axis_index("s") * NC + lax.axis_index("c")
        ibase = wid * IPT
        pltpu.sync_copy(idx_hbm.at[pl.ds(ibase, IPT)], idx_v)

        def start(chunk, buf):
            pltpu.async_copy(
                table_hbm.at[idx_v.at[pl.ds(chunk * CH_I, CH_I)]],
                rows_v.at[buf], sems[buf])

        def wait(chunk, buf):
            pltpu.make_async_copy(
                table_hbm.at[idx_v.at[pl.ds(chunk * CH_I, CH_I)]],
                rows_v.at[buf], sems[buf]).wait()

        for k in range(NBUF - 1):
            start(k, k)

        @pl.loop(0, NCHUNK, step=NBUF)
        def _(g):
            for k in range(NBUF):
                wait(g + k, k)
                nxt = g + k + (NBUF - 1)

                @pl.when(nxt < NCHUNK)
                def _(_nxt=nxt, _buf=(k + NBUF - 1) % NBUF):
                    start(_nxt, _buf)

                _reduce_chunk(rows_v.at[k], out_v, g + k)

        pltpu.sync_copy(out_v, out_hbm.at[pl.ds(wid * SPT, SPT)])

    packed = tile_kernel(flat_ids, table_pk)
    out16 = lax.bitcast_convert_type(packed, jnp.bfloat16).reshape(B, D)
    return out16.astype(jnp.float32)


# speedup vs baseline: 8.4942x; 8.4942x over previous
import functools

import jax
import jax.numpy as jnp
from jax import lax
from jax.experimental import pallas as pl
from jax.experimental.pallas import tpu as pltpu
from jax.experimental.pallas import tpu_sc as plsc

VOCAB = 100000
D = 128
B = 4096
L = 50

NC = 2
NS = 16
NW = NC * NS
LANES = 16
NCH = D // LANES

SPT = B // NW
IPT = SPT * L
CH_S = 4
CH_I = CH_S * L
NCHUNK = SPT // CH_S
NBUF = 4

_SCALE = 1.0 / L


def _reduce_chunk(rows_v, out_v, chunk):
    zero = jnp.zeros((LANES,), jnp.float32)
    for s in range(CH_S):
        row0 = s * L
        init = (tuple(zero for _ in range(NCH)), tuple(zero for _ in range(NCH)))

        @plsc.parallel_loop(0, L // 2, carry=init)
        def accs(i, carry, _row0=row0):
            a, b = carry
            ra = _row0 + 2 * i
            a = tuple(
                a[c] + rows_v[ra, pl.ds(c * LANES, LANES)] for c in range(NCH)
            )
            b = tuple(
                b[c] + rows_v[ra + 1, pl.ds(c * LANES, LANES)]
                for c in range(NCH)
            )
            return (a, b)

        a, b = accs
        orow = chunk * CH_S + s
        for c in range(NCH):
            out_v[orow, pl.ds(c * LANES, LANES)] = (a[c] + b[c]) * jnp.float32(
                _SCALE)


def kernel(token_ids, emb_table):
    flat_ids = token_ids.reshape(-1).astype(jnp.int32)
    mesh = plsc.VectorSubcoreMesh(core_axis_name="c", subcore_axis_name="s")

    @functools.partial(
        pl.kernel,
        out_type=jax.ShapeDtypeStruct((B, D), jnp.float32),
        mesh=mesh,
        scratch_types=[
            pltpu.VMEM((IPT,), jnp.int32),
            pltpu.VMEM((NBUF, CH_I, D), jnp.float32),
            pltpu.VMEM((SPT, D), jnp.float32),
        ]
        + [pltpu.SemaphoreType.DMA] * NBUF,
    )
    def tile_kernel(idx_hbm, table_hbm, out_hbm, idx_v, rows_v, out_v, *sems):
        wid = lax.axis_index("s") * NC + lax.axis_index("c")
        ibase = wid * IPT
        pltpu.sync_copy(idx_hbm.at[pl.ds(ibase, IPT)], idx_v)

        def start(chunk, buf):
            pltpu.async_copy(
                table_hbm.at[idx_v.at[pl.ds(chunk * CH_I, CH_I)]],
                rows_v.at[buf], sems[buf])

        def wait(chunk, buf):
            pltpu.make_async_copy(
                table_hbm.at[idx_v.at[pl.ds(chunk * CH_I, CH_I)]],
                rows_v.at[buf], sems[buf]).wait()

        for k in range(NBUF - 1):
            start(k, k)

        @pl.loop(0, NCHUNK, step=NBUF)
        def _(g):
            for k in range(NBUF):
                nxt = g + k + (NBUF - 1)

                @pl.when(nxt < NCHUNK)
                def _(_nxt=nxt, _buf=(k + NBUF - 1) % NBUF):
                    start(_nxt, _buf)

                wait(g + k, k)
                _reduce_chunk(rows_v.at[k], out_v, g + k)

        pltpu.sync_copy(out_v, out_hbm.at[pl.ds(wid * SPT, SPT)])

    return tile_kernel(flat_ids, emb_table)


# speedup vs baseline: 8.6695x; 1.0206x over previous
import functools

import jax
import jax.numpy as jnp
from jax import lax
from jax.experimental import pallas as pl
from jax.experimental.pallas import tpu as pltpu
from jax.experimental.pallas import tpu_sc as plsc

VOCAB = 100000
D = 128
B = 4096
L = 50

NC = 2
NS = 16
NW = NC * NS
LANES = 16
NCH = D // LANES

SPT = B // NW
IPT = SPT * L
CH_S = 4
CH_I = CH_S * L
NCHUNK = SPT // CH_S
NBUF = 4

_SCALE = 1.0 / L


def _reduce_chunk(rows_v, out_v, chunk):
    zero = jnp.zeros((LANES,), jnp.float32)
    for s in range(CH_S):
        row0 = s * L
        init = (tuple(zero for _ in range(NCH)), tuple(zero for _ in range(NCH)))

        @plsc.parallel_loop(0, L // 2, carry=init)
        def accs(i, carry, _row0=row0):
            a, b = carry
            ra = _row0 + 2 * i
            a = tuple(
                a[c] + rows_v[ra, pl.ds(c * LANES, LANES)] for c in range(NCH)
            )
            b = tuple(
                b[c] + rows_v[ra + 1, pl.ds(c * LANES, LANES)]
                for c in range(NCH)
            )
            return (a, b)

        a, b = accs
        orow = chunk * CH_S + s
        for c in range(NCH):
            out_v[orow, pl.ds(c * LANES, LANES)] = (a[c] + b[c]) * jnp.float32(
                _SCALE)


def kernel(token_ids, emb_table):
    flat_ids = token_ids.reshape(-1).astype(jnp.int32)
    mesh = plsc.VectorSubcoreMesh(core_axis_name="c", subcore_axis_name="s")

    @functools.partial(
        pl.kernel,
        out_type=jax.ShapeDtypeStruct((B, D), jnp.float32),
        mesh=mesh,
        scratch_types=[
            pltpu.VMEM((IPT,), jnp.int32),
            pltpu.VMEM((NBUF, CH_I, D), jnp.float32),
            pltpu.VMEM((SPT, D), jnp.float32),
        ]
        + [pltpu.SemaphoreType.DMA] * (2 * NBUF),
    )
    def tile_kernel(idx_hbm, table_hbm, out_hbm, idx_v, rows_v, out_v, *sems):
        wid = lax.axis_index("s") * NC + lax.axis_index("c")
        ibase = wid * IPT
        pltpu.sync_copy(idx_hbm.at[pl.ds(ibase, IPT)], idx_v)

        SPLIT = 96

        def start(chunk, buf):
            pltpu.async_copy(
                table_hbm.at[idx_v.at[pl.ds(chunk * CH_I, SPLIT)]],
                rows_v.at[buf].at[pl.ds(0, SPLIT)], sems[buf])
            pltpu.async_copy(
                table_hbm.at[idx_v.at[pl.ds(chunk * CH_I + SPLIT,
                                            CH_I - SPLIT)]],
                rows_v.at[buf].at[pl.ds(SPLIT, CH_I - SPLIT)],
                sems[NBUF + buf])

        def wait(chunk, buf):
            pltpu.make_async_copy(
                table_hbm.at[idx_v.at[pl.ds(chunk * CH_I, SPLIT)]],
                rows_v.at[buf].at[pl.ds(0, SPLIT)], sems[buf]).wait()
            pltpu.make_async_copy(
                table_hbm.at[idx_v.at[pl.ds(chunk * CH_I + SPLIT,
                                            CH_I - SPLIT)]],
                rows_v.at[buf].at[pl.ds(SPLIT, CH_I - SPLIT)],
                sems[NBUF + buf]).wait()

        for k in range(NBUF - 1):
            start(k, k)

        @pl.loop(0, NCHUNK, step=NBUF)
        def _(g):
            for k in range(NBUF):
                wait(g + k, k)
                nxt = g + k + (NBUF - 1)

                @pl.when(nxt < NCHUNK)
                def _(_nxt=nxt, _buf=(k + NBUF - 1) % NBUF):
                    start(_nxt, _buf)

                _reduce_chunk(rows_v.at[k], out_v, g + k)

        pltpu.sync_copy(out_v, out_hbm.at[pl.ds(wid * SPT, SPT)])

    return tile_kernel(flat_ids, emb_table)
